# baseline (speedup 1.0000x reference)
import jax
import jax.numpy as jnp
from jax import lax
from jax.experimental import pallas as pl
from jax.experimental.pallas import tpu as pltpu
from jax.experimental.pallas import tpu_sc as plsc

_B = 4
_D = 256
_H = 32
_W = 32
_NW = 32
_ROWS = _B * 2 * _D
_RPW = _ROWS // _NW


def _body(tab_hbm, out_hbm, tab_v, buf):
    wid = lax.axis_index("s") * 2 + lax.axis_index("c")
    row0 = wid * _RPW
    c0 = row0 % (2 * _D)
    is_col = c0 < _D

    pltpu.sync_copy(tab_hbm.at[pl.ds(c0, _RPW)], tab_v)

    def col_body(t, _):
        v0 = tab_v[t, pl.ds(0, 16)]
        v1 = tab_v[t, pl.ds(16, 16)]
        for i in range(_H):
            buf[t, pl.ds(i * _W, 16)] = v0
            buf[t, pl.ds(i * _W + 16, 16)] = v1
        return 0

    def row_body(t, _):
        a = tab_v[t, pl.ds(0, 16)]
        b = tab_v[t, pl.ds(16, 16)]
        for i in range(_H):
            s = a[i] if i < 16 else b[i - 16]
            v = jnp.full((16,), s, jnp.float32)
            buf[t, pl.ds(i * _W, 16)] = v
            buf[t, pl.ds(i * _W + 16, 16)] = v
        return 0

    @pl.when(is_col)
    def _():
        lax.fori_loop(0, _RPW, col_body, 0)

    @pl.when(jnp.logical_not(is_col))
    def _():
        lax.fori_loop(0, _RPW, row_body, 0)

    pltpu.sync_copy(buf, out_hbm.at[pl.ds(row0, _RPW)])


@jax.jit
def _pos_sc(tab):
    mesh = plsc.VectorSubcoreMesh(core_axis_name="c", subcore_axis_name="s")
    out = pl.kernel(
        _body,
        out_type=jax.ShapeDtypeStruct((_ROWS, _H * _W), jnp.float32),
        mesh=mesh,
        scratch_types=[
            pltpu.VMEM((_RPW, _H), jnp.float32),
            pltpu.VMEM((_RPW, _H * _W), jnp.float32),
        ],
    )(tab)
    return out.reshape(_B, 2 * _D, _H, _W)


def kernel(tensors, row_embed, col_embed):
    tab_t = jnp.concatenate([col_embed[:_W], row_embed[:_H]], axis=1).T
    return _pos_sc(tab_t)


# speedup vs baseline: 1.0484x; 1.0484x over previous
import jax
import jax.numpy as jnp
from jax import lax
from jax.experimental import pallas as pl
from jax.experimental.pallas import tpu as pltpu
from jax.experimental.pallas import tpu_sc as plsc

_B = 4
_D = 256
_H = 32
_W = 32
_NW = 32
_ROWS = _B * 2 * _D
_RPW = _ROWS // _NW
_CH = 8
_NCH = _RPW // _CH


def _body(tab_hbm, out_hbm, tab_v, buf0, buf1, sem0, sem1):
    wid = lax.axis_index("s") * 2 + lax.axis_index("c")
    row0 = wid * _RPW
    bb = row0 // (2 * _D)
    c0 = row0 % (2 * _D)
    is_col = c0 < _D

    pltpu.sync_copy(tab_hbm.at[pl.ds(c0, _RPW)], tab_v)

    def fill_col(buf, t, tl):
        v0 = tab_v[t, pl.ds(0, 16)]
        v1 = tab_v[t, pl.ds(16, 16)]
        for i in range(_H):
            buf[tl, i, pl.ds(0, 16)] = v0
            buf[tl, i, pl.ds(16, 16)] = v1

    def fill_row(buf, t, tl):
        a = tab_v[t, pl.ds(0, 16)]
        b = tab_v[t, pl.ds(16, 16)]
        for i in range(_H):
            s = a[i] if i < 16 else b[i - 16]
            v = jnp.full((16,), s, jnp.float32)
            buf[tl, i, pl.ds(0, 16)] = v
            buf[tl, i, pl.ds(16, 16)] = v

    def chunk_body(k, _):
        even = lax.rem(k, 2) == 0

        def run(buf, sem):
            @pl.when(k >= 2)
            def _():
                pltpu.make_async_copy(buf, out_hbm.at[bb, pl.ds(c0, _CH)], sem).wait()

            def fill(t, _):
                tl = lax.rem(t, _CH)

                @pl.when(is_col)
                def _():
                    fill_col(buf, t, tl)

                @pl.when(jnp.logical_not(is_col))
                def _():
                    fill_row(buf, t, tl)

                return 0

            lax.fori_loop(k * _CH, (k + 1) * _CH, fill, 0)
            pltpu.async_copy(buf, out_hbm.at[bb, pl.ds(c0 + k * _CH, _CH)], sem)

        @pl.when(even)
        def _():
            run(buf0, sem0)

        @pl.when(jnp.logical_not(even))
        def _():
            run(buf1, sem1)

        return 0

    lax.fori_loop(0, _NCH, chunk_body, 0)
    pltpu.make_async_copy(buf0, out_hbm.at[bb, pl.ds(c0, _CH)], sem0).wait()
    pltpu.make_async_copy(buf1, out_hbm.at[bb, pl.ds(c0, _CH)], sem1).wait()


@jax.jit
def _pos_sc(tab):
    mesh = plsc.VectorSubcoreMesh(core_axis_name="c", subcore_axis_name="s")
    return pl.kernel(
        _body,
        out_type=jax.ShapeDtypeStruct((_B, 2 * _D, _H, _W), jnp.float32),
        mesh=mesh,
        scratch_types=[
            pltpu.VMEM((_RPW, _H), jnp.float32),
            pltpu.VMEM((_CH, _H, _W), jnp.float32),
            pltpu.VMEM((_CH, _H, _W), jnp.float32),
            pltpu.SemaphoreType.DMA,
            pltpu.SemaphoreType.DMA,
        ],
    )(tab)


def kernel(tensors, row_embed, col_embed):
    tab_t = jnp.concatenate([col_embed[:_W], row_embed[:_H]], axis=1).T
    return _pos_sc(tab_t)


# speedup vs baseline: 2.0508x; 1.9561x over previous
import jax
import jax.numpy as jnp
from jax import lax
from jax.experimental import pallas as pl
from jax.experimental.pallas import tpu as pltpu
from jax.experimental.pallas import tpu_sc as plsc

_B = 4
_D = 256
_H = 32
_W = 32
_ROWS_PER_B = _H * _W


def _body(ce_hbm, re_hbm, out_hbm, ce_v, re_v, buf, sem):
    wid = lax.axis_index("s") * 2 + lax.axis_index("c")

    pltpu.sync_copy(ce_hbm, ce_v)
    pltpu.sync_copy(re_hbm.at[pl.ds(wid, 1)], re_v)

    rv = [re_v[0, pl.ds(k * 16, 16)] for k in range(_D // 16)]

    def fill(j, _):
        for k in range(_D // 16):
            buf[j, pl.ds(k * 16, 16)] = ce_v[j, pl.ds(k * 16, 16)]
        for k in range(_D // 16):
            buf[j, pl.ds(_D + k * 16, 16)] = rv[k]
        return 0

    lax.fori_loop(0, _W, fill, 0)

    for b in range(_B):
        pltpu.async_copy(buf, out_hbm.at[b, pl.ds(wid * _W, _W)], sem)
    for b in range(_B):
        pltpu.make_async_copy(buf, out_hbm.at[b, pl.ds(wid * _W, _W)], sem).wait()


@jax.jit
def _pos_sc(ce, re):
    mesh = plsc.VectorSubcoreMesh(core_axis_name="c", subcore_axis_name="s")
    pk = pl.kernel(
        _body,
        out_type=jax.ShapeDtypeStruct((_B, _ROWS_PER_B, 2 * _D), jnp.float32),
        mesh=mesh,
        scratch_types=[
            pltpu.VMEM((_H, _D), jnp.float32),
            pltpu.VMEM((1, _D), jnp.float32),
            pltpu.VMEM((_W, 2 * _D), jnp.float32),
            pltpu.SemaphoreType.DMA,
        ],
    )(ce, re)
    return jnp.transpose(pk.reshape(_B, _H, _W, 2 * _D), (0, 3, 1, 2))


def kernel(tensors, row_embed, col_embed):
    return _pos_sc(col_embed[:_W], row_embed[:_H])
